# baseline (speedup 1.0000x reference)
import functools

import jax
import jax.numpy as jnp
from jax import lax
from jax.experimental import pallas as pl
from jax.experimental.pallas import tpu as pltpu
from jax.experimental.pallas import tpu_sc as plsc

EPS = 1e-5
B, C, N, D = 4, 3, 1024, 768
K = 16
KN = K - 1
TQ = 256
TN = 128
NW = 32
QW = (B * N) // NW
QL = 16
NCH = QW // QL
NBLK = N // 16
CNT = float(B * N * KN)

_HI = jax.lax.Precision.HIGHEST


def _dot(a, b, dims):
    return lax.dot_general(a, b, (dims, ((), ())),
                           preferred_element_type=jnp.float32, precision=_HI)


def _dotd(a, b, dims):
    return lax.dot_general(a, b, (dims, ((), ())),
                           preferred_element_type=jnp.float32,
                           precision=jax.lax.Precision.DEFAULT)


def _ka_body(x_ref, w0_ref, b0_ref, w1_ref, b1_ref, pe_ref, xh_ref, nm_ref):
    b = pl.program_id(0)
    t = pl.program_id(1)
    xall = x_ref[...]
    m0 = jnp.mean(xall, axis=(0, 2), keepdims=True)
    v0 = jnp.mean((xall - m0) ** 2, axis=(0, 2), keepdims=True)
    den = jnp.sqrt(v0[0] + EPS)
    m3 = m0[0]
    w0 = w0_ref[...]
    b0 = b0_ref[...]
    xb = x_ref[b]
    xbh = (xb - m3) / den * w0 + b0
    xb_t = x_ref[b, :, pl.ds(t * TQ, TQ)]
    xbh_t = (xb_t - m3) / den * w0 + b0
    xh_ref[0] = xbh_t

    pe_ref[0] = _dotd(w1_ref[...], xbh_t, ((1,), (0,))) + b1_ref[...]

    ones_c = jnp.ones((C, 1), jnp.float32)
    xs_c = _dot(xbh * xbh, ones_c, ((0,), (0,)))
    xs_q = _dot(ones_c, xbh_t * xbh_t, ((0,), (0,)))
    inner = _dotd(xbh, xbh_t, ((0,), (0,)))
    d2 = jnp.maximum(xs_c + xs_q - 2.0 * inner, 0.0)
    nm = jnp.sqrt(d2)
    for g in range(TQ // 16):
        nm_ref[0, g] = nm[:, g * 16:(g + 1) * 16]


def _sc_body(nm_ref, xf_ref, nb_ref, mom_ref, xv, dbuf, bmaxv, idxv, nbv,
             momv):
    ci = lax.axis_index("c")
    si = lax.axis_index("s")
    wid = si * 2 + ci
    b = wid // (N // TN)
    t = wid % (N // TN)
    pltpu.sync_copy(xf_ref, xv)
    xbase = b * (C * N)
    lanes = lax.broadcasted_iota(jnp.int32, (16,), 0)
    rowbase = (b * N + t * QW) * N

    def chunk_body(ch, carry):
        pltpu.sync_copy(nm_ref.at[pl.ds(rowbase + ch * (QL * N), QL * N)],
                        dbuf)

        def bmax_body(blk, _):
            acc = dbuf[pl.ds(blk * 256, 16)]
            for u in range(1, 16):
                acc = jnp.maximum(acc, dbuf[pl.ds(blk * 256 + u * 16, 16)])
            bmaxv[blk] = acc
            return 0

        lax.fori_loop(0, NBLK, bmax_body, 0)

        def top_body(j, _):
            gmax = bmaxv[0]
            for i in range(1, NBLK):
                gmax = jnp.maximum(gmax, bmaxv[i])
            blk = jnp.full((16,), NBLK, jnp.int32)
            for i in range(NBLK - 1, -1, -1):
                blk = jnp.where(bmaxv[i] == gmax, i, blk)
            base = blk * 256
            pos = jnp.full((16,), N, jnp.int32)
            for u in range(16):
                v = plsc.load_gather(dbuf, [base + u * 16 + lanes])
                pos = jnp.minimum(pos,
                                  jnp.where(v == gmax, blk * 16 + u, N))
            idxv[j] = pos
            plsc.store_scatter(dbuf, [pos * 16 + lanes],
                               jnp.full((16,), -1.0, jnp.float32))
            macc = jnp.full((16,), -2.0, jnp.float32)
            for u in range(16):
                macc = jnp.maximum(
                    macc, plsc.load_gather(dbuf, [base + u * 16 + lanes]))
            plsc.store_scatter(bmaxv, [blk, lanes], macc)
            return 0

        lax.fori_loop(0, K, top_body, 0)

        s0, s1a, s2, m00, m01, m02, m11, m12, m22 = carry
        qoff = t * QW + ch * QL
        for j in range(1, K):
            sel = idxv[j]
            nbs = []
            for c in range(C):
                src = plsc.load_gather(xv, [xbase + c * N + sel])
                ctr = xv[pl.ds(xbase + c * N + qoff, QL)]
                nbc = src - ctr
                nbs.append(nbc)
                nbv[c, j - 1, pl.ds(ch * QL, QL)] = nbc
            s0 = s0 + nbs[0]
            s1a = s1a + nbs[1]
            s2 = s2 + nbs[2]
            m00 = m00 + nbs[0] * nbs[0]
            m01 = m01 + nbs[0] * nbs[1]
            m02 = m02 + nbs[0] * nbs[2]
            m11 = m11 + nbs[1] * nbs[1]
            m12 = m12 + nbs[1] * nbs[2]
            m22 = m22 + nbs[2] * nbs[2]
        return (s0, s1a, s2, m00, m01, m02, m11, m12, m22)

    zero = jnp.zeros((16,), jnp.float32)
    carry = lax.fori_loop(0, NCH, chunk_body, (zero,) * 9)
    s0, s1a, s2, m00, m01, m02, m11, m12, m22 = carry

    def _row(a, bq, cq, dq):
        r = jnp.where(lanes == 0, jnp.sum(a), 0.0)
        r = jnp.where(lanes == 1, jnp.sum(bq), r)
        r = jnp.where(lanes == 2, jnp.sum(cq), r)
        return jnp.where(lanes == 3, jnp.sum(dq), r)

    momv[0] = _row(s0, m00, m01, m02)
    momv[1] = _row(s1a, m01, m11, m12)
    momv[2] = _row(s2, m02, m12, m22)
    pltpu.sync_copy(nbv, nb_ref.at[wid])
    pltpu.sync_copy(momv, mom_ref.at[wid])


def _kb_body(nb_ref, pe_ref, mom_ref, w2a_ref, a_w_ref, a_b_ref,
             b_w_ref, b_b_ref, w2b_ref, fp_ref, fn_ref, fb_ref, out_ref):
    msum = jnp.sum(mom_ref[...], axis=0)
    s_sum = msum[:, 0:1]
    m_sum = msum[:, 1:1 + C]
    mean_n = s_sum / CNT
    eye = (lax.broadcasted_iota(jnp.int32, (C, C), 0) ==
           lax.broadcasted_iota(jnp.int32, (C, C), 1)).astype(jnp.float32)
    diag = jnp.sum(m_sum * eye, axis=1, keepdims=True)
    var_n = diag / CNT - mean_n * mean_n
    s_col = a_w_ref[...] / jnp.sqrt(var_n + EPS)
    t_col = a_b_ref[...] - mean_n * s_col

    w2a = w2a_ref[...]
    s_row = _dot(s_col, eye, ((0,), (0,)))
    mean_row = _dot(mean_n, eye, ((0,), (0,)))
    w2a_s = w2a * s_row
    mean_y = _dot(w2a, s_col * mean_n + t_col, ((1,), (0,)))
    cov = m_sum / CNT - _dot(mean_n, mean_row, ((1,), (0,)))
    v_half = _dot(w2a_s, cov, ((1,), (0,)))
    var_y = jnp.sum(v_half * w2a_s, axis=1, keepdims=True)
    inv_y = 1.0 / jnp.sqrt(var_y + EPS)
    b_w = b_w_ref[...]
    b_b = b_b_ref[...]

    nb_all = jnp.concatenate([nb_ref[0, :, kk, :] for kk in range(KN)],
                             axis=1)
    xn = nb_all * s_col + t_col
    y = _dotd(w2a, xn, ((1,), (0,)))
    z = (y - mean_y) * inv_y * b_w + b_b
    g = 0.5 * z * (1.0 + lax.erf(z * (2.0 ** -0.5)))
    h2 = _dotd(w2b_ref[...], g, ((1,), (0,)))
    acc = h2[:, :TN]
    for kk in range(1, KN):
        acc = jnp.maximum(acc, h2[:, kk * TN:(kk + 1) * TN])

    out = (_dotd(fp_ref[...], pe_ref[0], ((1,), (0,))) +
           _dotd(fn_ref[...], acc, ((1,), (0,))) + fb_ref[...])
    out_ref[0] = out


@functools.partial(jax.jit, static_argnames=())
def kernel(x, bn0_w, bn0_b, conv1_w, conv1_b, bn2a_w, bn2a_b, conv2a_w,
           bn2b_w, bn2b_b, conv2b_w, final_w, final_b):
    f32 = jnp.float32
    w0 = bn0_w.reshape(C, 1).astype(f32)
    b0 = bn0_b.reshape(C, 1).astype(f32)
    b1 = conv1_b.reshape(D, 1).astype(f32)
    a_w = bn2a_w.reshape(C, 1).astype(f32)
    a_b = bn2a_b.reshape(C, 1).astype(f32)
    b_w = bn2b_w.reshape(D, 1).astype(f32)
    b_b = bn2b_b.reshape(D, 1).astype(f32)
    fb = final_b.reshape(D, 1).astype(f32)
    fp = final_w[:, :D]
    fn = final_w[:, D:]

    n_tq = N // TQ
    pe, xh, nm = pl.pallas_call(
        _ka_body,
        grid=(B, n_tq),
        in_specs=[
            pl.BlockSpec((B, C, N), lambda b, t: (0, 0, 0)),
            pl.BlockSpec((C, 1), lambda b, t: (0, 0)),
            pl.BlockSpec((C, 1), lambda b, t: (0, 0)),
            pl.BlockSpec((D, C), lambda b, t: (0, 0)),
            pl.BlockSpec((D, 1), lambda b, t: (0, 0)),
        ],
        out_specs=[
            pl.BlockSpec((1, D, TQ), lambda b, t: (b, 0, t)),
            pl.BlockSpec((1, C, TQ), lambda b, t: (b, 0, t)),
            pl.BlockSpec((1, TQ // 16, N, 16), lambda b, t: (b, t, 0, 0)),
        ],
        out_shape=[
            jax.ShapeDtypeStruct((B, D, N), f32),
            jax.ShapeDtypeStruct((B, C, N), f32),
            jax.ShapeDtypeStruct((B, N // 16, N, 16), f32),
        ],
    )(x, w0, b0, conv1_w, b1)

    mesh = plsc.VectorSubcoreMesh(core_axis_name="c", subcore_axis_name="s")
    nb, mom = pl.kernel(
        _sc_body,
        out_type=[
            jax.ShapeDtypeStruct((NW, C, KN, TN), f32),
            jax.ShapeDtypeStruct((NW, C, 16), f32),
        ],
        mesh=mesh,
        compiler_params=pltpu.CompilerParams(needs_layout_passes=False,
                                             use_tc_tiling_on_sc=False),
        scratch_types=[
            pltpu.VMEM((B * C * N,), f32),
            pltpu.VMEM((QL * N,), f32),
            pltpu.VMEM((NBLK, 16), f32),
            pltpu.VMEM((K, 16), jnp.int32),
            pltpu.VMEM((C, KN, TN), f32),
            pltpu.VMEM((C, 16), f32),
        ],
    )(nm.reshape(B * N * N), xh.reshape(B * C * N))

    n_tn = N // TN
    out = pl.pallas_call(
        _kb_body,
        grid=(B, n_tn),
        in_specs=[
            pl.BlockSpec((1, C, KN, TN), lambda b, t: (b * (N // TN) + t,
                                                       0, 0, 0)),
            pl.BlockSpec((1, D, TN), lambda b, t: (b, 0, t)),
            pl.BlockSpec((NW, C, 16), lambda b, t: (0, 0, 0)),
            pl.BlockSpec((D, C), lambda b, t: (0, 0)),
            pl.BlockSpec((C, 1), lambda b, t: (0, 0)),
            pl.BlockSpec((C, 1), lambda b, t: (0, 0)),
            pl.BlockSpec((D, 1), lambda b, t: (0, 0)),
            pl.BlockSpec((D, 1), lambda b, t: (0, 0)),
            pl.BlockSpec((D, D), lambda b, t: (0, 0)),
            pl.BlockSpec((D, D), lambda b, t: (0, 0)),
            pl.BlockSpec((D, D), lambda b, t: (0, 0)),
            pl.BlockSpec((D, 1), lambda b, t: (0, 0)),
        ],
        out_specs=pl.BlockSpec((1, D, TN), lambda b, t: (b, 0, t)),
        out_shape=jax.ShapeDtypeStruct((B, D, N), f32),
    )(nb, pe, mom, conv2a_w, a_w, a_b, b_w, b_b, conv2b_w, fp, fn, fb)
    return out


# speedup vs baseline: 1.2978x; 1.2978x over previous
import functools

import jax
import jax.numpy as jnp
from jax import lax
from jax.experimental import pallas as pl
from jax.experimental.pallas import tpu as pltpu
from jax.experimental.pallas import tpu_sc as plsc

EPS = 1e-5
B, C, N, D = 4, 3, 1024, 768
K = 16
KN = K - 1
TQ = 256
TN = 128
NW = 32
QW = (B * N) // NW
QL = 16
NCH = QW // QL
NBLK = N // 16
CNT = float(B * N * KN)

_HI = jax.lax.Precision.HIGHEST


def _dot(a, b, dims):
    return lax.dot_general(a, b, (dims, ((), ())),
                           preferred_element_type=jnp.float32, precision=_HI)


def _dotd(a, b, dims):
    return lax.dot_general(a, b, (dims, ((), ())),
                           preferred_element_type=jnp.float32,
                           precision=jax.lax.Precision.DEFAULT)


def _ka_body(x_ref, w0_ref, b0_ref, w1_ref, b1_ref, pe_ref, xh_ref, nm_ref):
    b = pl.program_id(0)
    t = pl.program_id(1)
    xall = x_ref[...]
    m0 = jnp.mean(xall, axis=(0, 2), keepdims=True)
    v0 = jnp.mean((xall - m0) ** 2, axis=(0, 2), keepdims=True)
    den = jnp.sqrt(v0[0] + EPS)
    m3 = m0[0]
    w0 = w0_ref[...]
    b0 = b0_ref[...]
    xb = x_ref[b]
    xbh = (xb - m3) / den * w0 + b0
    xb_t = x_ref[b, :, pl.ds(t * TQ, TQ)]
    xbh_t = (xb_t - m3) / den * w0 + b0
    xh_ref[0] = xbh_t

    pe_ref[0] = _dotd(w1_ref[...], xbh_t, ((1,), (0,))) + b1_ref[...]

    ones_c = jnp.ones((C, 1), jnp.float32)
    xs_q = _dot(xbh_t * xbh_t, ones_c, ((0,), (0,)))
    xs_c = _dot(ones_c, xbh * xbh, ((0,), (0,)))
    inner = _dotd(xbh_t, xbh, ((0,), (0,)))
    d2 = jnp.maximum(xs_q + xs_c - 2.0 * inner, 0.0)
    nm_ref[0] = jnp.sqrt(d2)


def _sc_body(nm_ref, xf_ref, nb_ref, mom_ref, xv, dbuf, bmaxv, idxv, nbv,
             momv):
    ci = lax.axis_index("c")
    si = lax.axis_index("s")
    wid = si * 2 + ci
    b = wid // (N // TN)
    t = wid % (N // TN)
    pltpu.sync_copy(xf_ref, xv)
    xbase = b * (C * N)
    lanes = lax.broadcasted_iota(jnp.int32, (16,), 0)
    laneoff = lanes * N
    rowbase = (b * N + t * QW) * N
    rots = [(lanes + u) & 15 for u in range(16)]

    def chunk_body(ch, carry):
        pltpu.sync_copy(nm_ref.at[pl.ds(rowbase + ch * (QL * N), QL * N)],
                        dbuf)

        def bmax_body(blk, _):
            acc = plsc.load_gather(dbuf, [laneoff + blk * 16 + rots[0]])
            for u in range(1, 16):
                acc = jnp.maximum(
                    acc,
                    plsc.load_gather(dbuf, [laneoff + blk * 16 + rots[u]]))
            bmaxv[blk] = acc
            return 0

        lax.fori_loop(0, NBLK, bmax_body, 0)

        def top_body(j, _):
            gmax = bmaxv[0]
            for i in range(1, NBLK):
                gmax = jnp.maximum(gmax, bmaxv[i])
            blk = jnp.full((16,), NBLK, jnp.int32)
            for i in range(NBLK - 1, -1, -1):
                blk = jnp.where(bmaxv[i] == gmax, i, blk)
            base16 = blk * 16
            pos = jnp.full((16,), N, jnp.int32)
            for u in range(16):
                v = plsc.load_gather(dbuf, [laneoff + base16 + rots[u]])
                pos = jnp.minimum(pos,
                                  jnp.where(v == gmax, base16 + rots[u], N))
            idxv[j] = pos
            plsc.store_scatter(dbuf, [laneoff + pos],
                               jnp.full((16,), -1.0, jnp.float32))
            macc = jnp.full((16,), -2.0, jnp.float32)
            for u in range(16):
                macc = jnp.maximum(
                    macc,
                    plsc.load_gather(dbuf, [laneoff + base16 + rots[u]]))
            plsc.store_scatter(bmaxv, [blk, lanes], macc)
            return 0

        lax.fori_loop(0, K, top_body, 0)

        s0, s1a, s2, m00, m01, m02, m11, m12, m22 = carry
        qoff = t * QW + ch * QL
        for j in range(1, K):
            sel = idxv[j]
            nbs = []
            for c in range(C):
                src = plsc.load_gather(xv, [xbase + c * N + sel])
                ctr = xv[pl.ds(xbase + c * N + qoff, QL)]
                nbc = src - ctr
                nbs.append(nbc)
                nbv[c, j - 1, pl.ds(ch * QL, QL)] = nbc
            s0 = s0 + nbs[0]
            s1a = s1a + nbs[1]
            s2 = s2 + nbs[2]
            m00 = m00 + nbs[0] * nbs[0]
            m01 = m01 + nbs[0] * nbs[1]
            m02 = m02 + nbs[0] * nbs[2]
            m11 = m11 + nbs[1] * nbs[1]
            m12 = m12 + nbs[1] * nbs[2]
            m22 = m22 + nbs[2] * nbs[2]
        return (s0, s1a, s2, m00, m01, m02, m11, m12, m22)

    zero = jnp.zeros((16,), jnp.float32)
    carry = lax.fori_loop(0, NCH, chunk_body, (zero,) * 9)
    s0, s1a, s2, m00, m01, m02, m11, m12, m22 = carry

    def _row(a, bq, cq, dq):
        r = jnp.where(lanes == 0, jnp.sum(a), 0.0)
        r = jnp.where(lanes == 1, jnp.sum(bq), r)
        r = jnp.where(lanes == 2, jnp.sum(cq), r)
        return jnp.where(lanes == 3, jnp.sum(dq), r)

    momv[0] = _row(s0, m00, m01, m02)
    momv[1] = _row(s1a, m01, m11, m12)
    momv[2] = _row(s2, m02, m12, m22)
    pltpu.sync_copy(nbv, nb_ref.at[wid])
    pltpu.sync_copy(momv, mom_ref.at[wid])


def _kb_body(nb_ref, pe_ref, mom_ref, w2a_ref, a_w_ref, a_b_ref,
             b_w_ref, b_b_ref, w2b_ref, fp_ref, fn_ref, fb_ref, out_ref):
    msum = jnp.sum(mom_ref[...], axis=0)
    s_sum = msum[:, 0:1]
    m_sum = msum[:, 1:1 + C]
    mean_n = s_sum / CNT
    eye = (lax.broadcasted_iota(jnp.int32, (C, C), 0) ==
           lax.broadcasted_iota(jnp.int32, (C, C), 1)).astype(jnp.float32)
    diag = jnp.sum(m_sum * eye, axis=1, keepdims=True)
    var_n = diag / CNT - mean_n * mean_n
    s_col = a_w_ref[...] / jnp.sqrt(var_n + EPS)
    t_col = a_b_ref[...] - mean_n * s_col

    w2a = w2a_ref[...]
    s_row = _dot(s_col, eye, ((0,), (0,)))
    mean_row = _dot(mean_n, eye, ((0,), (0,)))
    w2a_s = w2a * s_row
    mean_y = _dot(w2a, s_col * mean_n + t_col, ((1,), (0,)))
    cov = m_sum / CNT - _dot(mean_n, mean_row, ((1,), (0,)))
    v_half = _dot(w2a_s, cov, ((1,), (0,)))
    var_y = jnp.sum(v_half * w2a_s, axis=1, keepdims=True)
    inv_y = 1.0 / jnp.sqrt(var_y + EPS)
    b_w = b_w_ref[...]
    b_b = b_b_ref[...]

    nb_all = jnp.concatenate([nb_ref[0, :, kk, :] for kk in range(KN)],
                             axis=1)
    xn = nb_all * s_col + t_col
    y = _dotd(w2a, xn, ((1,), (0,)))
    z = (y - mean_y) * inv_y * b_w + b_b
    g = 0.5 * z * (1.0 + lax.erf(z * (2.0 ** -0.5)))
    h2 = _dotd(w2b_ref[...], g, ((1,), (0,)))
    acc = h2[:, :TN]
    for kk in range(1, KN):
        acc = jnp.maximum(acc, h2[:, kk * TN:(kk + 1) * TN])

    out = (_dotd(fp_ref[...], pe_ref[0], ((1,), (0,))) +
           _dotd(fn_ref[...], acc, ((1,), (0,))) + fb_ref[...])
    out_ref[0] = out


@functools.partial(jax.jit, static_argnames=())
def kernel(x, bn0_w, bn0_b, conv1_w, conv1_b, bn2a_w, bn2a_b, conv2a_w,
           bn2b_w, bn2b_b, conv2b_w, final_w, final_b):
    f32 = jnp.float32
    w0 = bn0_w.reshape(C, 1).astype(f32)
    b0 = bn0_b.reshape(C, 1).astype(f32)
    b1 = conv1_b.reshape(D, 1).astype(f32)
    a_w = bn2a_w.reshape(C, 1).astype(f32)
    a_b = bn2a_b.reshape(C, 1).astype(f32)
    b_w = bn2b_w.reshape(D, 1).astype(f32)
    b_b = bn2b_b.reshape(D, 1).astype(f32)
    fb = final_b.reshape(D, 1).astype(f32)
    fp = final_w[:, :D]
    fn = final_w[:, D:]

    n_tq = N // TQ
    pe, xh, nm = pl.pallas_call(
        _ka_body,
        grid=(B, n_tq),
        in_specs=[
            pl.BlockSpec((B, C, N), lambda b, t: (0, 0, 0)),
            pl.BlockSpec((C, 1), lambda b, t: (0, 0)),
            pl.BlockSpec((C, 1), lambda b, t: (0, 0)),
            pl.BlockSpec((D, C), lambda b, t: (0, 0)),
            pl.BlockSpec((D, 1), lambda b, t: (0, 0)),
        ],
        out_specs=[
            pl.BlockSpec((1, D, TQ), lambda b, t: (b, 0, t)),
            pl.BlockSpec((1, C, TQ), lambda b, t: (b, 0, t)),
            pl.BlockSpec((1, TQ, N), lambda b, t: (b, t, 0)),
        ],
        out_shape=[
            jax.ShapeDtypeStruct((B, D, N), f32),
            jax.ShapeDtypeStruct((B, C, N), f32),
            jax.ShapeDtypeStruct((B, N, N), f32),
        ],
    )(x, w0, b0, conv1_w, b1)

    mesh = plsc.VectorSubcoreMesh(core_axis_name="c", subcore_axis_name="s")
    nb, mom = pl.kernel(
        _sc_body,
        out_type=[
            jax.ShapeDtypeStruct((NW, C, KN, TN), f32),
            jax.ShapeDtypeStruct((NW, C, 16), f32),
        ],
        mesh=mesh,
        compiler_params=pltpu.CompilerParams(needs_layout_passes=False,
                                             use_tc_tiling_on_sc=False),
        scratch_types=[
            pltpu.VMEM((B * C * N,), f32),
            pltpu.VMEM((QL * N,), f32),
            pltpu.VMEM((NBLK, 16), f32),
            pltpu.VMEM((K, 16), jnp.int32),
            pltpu.VMEM((C, KN, TN), f32),
            pltpu.VMEM((C, 16), f32),
        ],
    )(nm.reshape(B * N * N), xh.reshape(B * C * N))

    n_tn = N // TN
    out = pl.pallas_call(
        _kb_body,
        grid=(B, n_tn),
        in_specs=[
            pl.BlockSpec((1, C, KN, TN), lambda b, t: (b * (N // TN) + t,
                                                       0, 0, 0)),
            pl.BlockSpec((1, D, TN), lambda b, t: (b, 0, t)),
            pl.BlockSpec((NW, C, 16), lambda b, t: (0, 0, 0)),
            pl.BlockSpec((D, C), lambda b, t: (0, 0)),
            pl.BlockSpec((C, 1), lambda b, t: (0, 0)),
            pl.BlockSpec((C, 1), lambda b, t: (0, 0)),
            pl.BlockSpec((D, 1), lambda b, t: (0, 0)),
            pl.BlockSpec((D, 1), lambda b, t: (0, 0)),
            pl.BlockSpec((D, D), lambda b, t: (0, 0)),
            pl.BlockSpec((D, D), lambda b, t: (0, 0)),
            pl.BlockSpec((D, D), lambda b, t: (0, 0)),
            pl.BlockSpec((D, 1), lambda b, t: (0, 0)),
        ],
        out_specs=pl.BlockSpec((1, D, TN), lambda b, t: (b, 0, t)),
        out_shape=jax.ShapeDtypeStruct((B, D, N), f32),
    )(nb, pe, mom, conv2a_w, a_w, a_b, b_w, b_b, conv2b_w, fp, fn, fb)
    return out


# speedup vs baseline: 1.4028x; 1.0809x over previous
import functools

import jax
import jax.numpy as jnp
from jax import lax
from jax.experimental import pallas as pl
from jax.experimental.pallas import tpu as pltpu
from jax.experimental.pallas import tpu_sc as plsc

EPS = 1e-5
B, C, N, D = 4, 3, 1024, 768
K = 16
KN = K - 1
TQ = 256
TN = 256
TNW = 128
NW = 32
QW = (B * N) // NW
QL = 16
NCH = QW // QL
NBLK = N // 16
CNT = float(B * N * KN)

_HI = jax.lax.Precision.HIGHEST


def _dot(a, b, dims):
    return lax.dot_general(a, b, (dims, ((), ())),
                           preferred_element_type=jnp.float32, precision=_HI)


def _dotd(a, b, dims):
    return lax.dot_general(a, b, (dims, ((), ())),
                           preferred_element_type=jnp.float32,
                           precision=jax.lax.Precision.DEFAULT)


def _ka_body(x_ref, w0_ref, b0_ref, w1_ref, b1_ref, pe_ref, xh_ref, nm_ref):
    b = pl.program_id(0)
    t = pl.program_id(1)
    xall = x_ref[...]
    m0 = jnp.mean(xall, axis=(0, 2), keepdims=True)
    v0 = jnp.mean((xall - m0) ** 2, axis=(0, 2), keepdims=True)
    den = jnp.sqrt(v0[0] + EPS)
    m3 = m0[0]
    w0 = w0_ref[...]
    b0 = b0_ref[...]
    xb = x_ref[b]
    xbh = (xb - m3) / den * w0 + b0
    xb_t = x_ref[b, :, pl.ds(t * TQ, TQ)]
    xbh_t = (xb_t - m3) / den * w0 + b0
    xh_ref[0] = xbh_t

    pe_ref[0] = _dotd(w1_ref[...], xbh_t, ((1,), (0,))) + b1_ref[...]

    ones_c = jnp.ones((C, 1), jnp.float32)
    xs_q = _dot(xbh_t * xbh_t, ones_c, ((0,), (0,)))
    xs_c = _dot(ones_c, xbh * xbh, ((0,), (0,)))
    inner = _dotd(xbh_t, xbh, ((0,), (0,)))
    d2 = jnp.maximum(xs_q + xs_c - 2.0 * inner, 0.0)
    nm_ref[0] = jnp.sqrt(d2)


def _sc_body(nm_ref, xf_ref, nb_ref, mom_ref, xv, dbuf, bmaxv, idxv, nbv,
             momv):
    ci = lax.axis_index("c")
    si = lax.axis_index("s")
    wid = si * 2 + ci
    b = wid // (N // TN)
    t = wid % (N // TN)
    pltpu.sync_copy(xf_ref, xv)
    xbase = b * (C * N)
    lanes = lax.broadcasted_iota(jnp.int32, (16,), 0)
    laneoff = lanes * N
    rowbase = (b * N + t * QW) * N
    rots = [(lanes + u) & 15 for u in range(16)]

    def chunk_body(ch, carry):
        pltpu.sync_copy(nm_ref.at[pl.ds(rowbase + ch * (QL * N), QL * N)],
                        dbuf)

        def bmax_body(blk, _):
            acc = plsc.load_gather(dbuf, [laneoff + blk * 16 + rots[0]])
            for u in range(1, 16):
                acc = jnp.maximum(
                    acc,
                    plsc.load_gather(dbuf, [laneoff + blk * 16 + rots[u]]))
            bmaxv[blk] = acc
            return 0

        lax.fori_loop(0, NBLK, bmax_body, 0)

        def top_body(j, _):
            gmax = bmaxv[0]
            for i in range(1, NBLK):
                gmax = jnp.maximum(gmax, bmaxv[i])
            blk = jnp.full((16,), NBLK, jnp.int32)
            for i in range(NBLK - 1, -1, -1):
                blk = jnp.where(bmaxv[i] == gmax, i, blk)
            base16 = blk * 16
            pos = jnp.full((16,), N, jnp.int32)
            vs = []
            for u in range(16):
                v = plsc.load_gather(dbuf, [laneoff + base16 + rots[u]])
                vs.append(v)
                pos = jnp.minimum(pos,
                                  jnp.where(v == gmax, base16 + rots[u], N))
            idxv[j] = pos
            plsc.store_scatter(dbuf, [laneoff + pos],
                               jnp.full((16,), -1.0, jnp.float32))
            macc = jnp.full((16,), -2.0, jnp.float32)
            for u in range(16):
                macc = jnp.maximum(
                    macc, jnp.where(base16 + rots[u] == pos, -1.0, vs[u]))
            plsc.store_scatter(bmaxv, [blk, lanes], macc)
            return 0

        lax.fori_loop(0, K, top_body, 0)

        s0, s1a, s2, m00, m01, m02, m11, m12, m22 = carry
        qoff = t * QW + ch * QL
        for j in range(1, K):
            sel = idxv[j]
            nbs = []
            for c in range(C):
                src = plsc.load_gather(xv, [xbase + c * N + sel])
                ctr = xv[pl.ds(xbase + c * N + qoff, QL)]
                nbc = src - ctr
                nbs.append(nbc)
                nbv[c, j - 1, pl.ds(ch * QL, QL)] = nbc
            s0 = s0 + nbs[0]
            s1a = s1a + nbs[1]
            s2 = s2 + nbs[2]
            m00 = m00 + nbs[0] * nbs[0]
            m01 = m01 + nbs[0] * nbs[1]
            m02 = m02 + nbs[0] * nbs[2]
            m11 = m11 + nbs[1] * nbs[1]
            m12 = m12 + nbs[1] * nbs[2]
            m22 = m22 + nbs[2] * nbs[2]
        return (s0, s1a, s2, m00, m01, m02, m11, m12, m22)

    zero = jnp.zeros((16,), jnp.float32)
    carry = lax.fori_loop(0, NCH, chunk_body, (zero,) * 9)
    s0, s1a, s2, m00, m01, m02, m11, m12, m22 = carry

    def _row(a, bq, cq, dq):
        r = jnp.where(lanes == 0, jnp.sum(a), 0.0)
        r = jnp.where(lanes == 1, jnp.sum(bq), r)
        r = jnp.where(lanes == 2, jnp.sum(cq), r)
        return jnp.where(lanes == 3, jnp.sum(dq), r)

    momv[0] = _row(s0, m00, m01, m02)
    momv[1] = _row(s1a, m01, m11, m12)
    momv[2] = _row(s2, m02, m12, m22)
    pltpu.sync_copy(nbv, nb_ref.at[wid])
    pltpu.sync_copy(momv, mom_ref.at[wid])


def _kb_body(nb_ref, pe_ref, mom_ref, w2a_ref, a_w_ref, a_b_ref,
             b_w_ref, b_b_ref, w2b_ref, fp_ref, fn_ref, fb_ref, out_ref):
    msum = jnp.sum(mom_ref[...], axis=0)
    s_sum = msum[:, 0:1]
    m_sum = msum[:, 1:1 + C]
    mean_n = s_sum / CNT
    eye = (lax.broadcasted_iota(jnp.int32, (C, C), 0) ==
           lax.broadcasted_iota(jnp.int32, (C, C), 1)).astype(jnp.float32)
    diag = jnp.sum(m_sum * eye, axis=1, keepdims=True)
    var_n = diag / CNT - mean_n * mean_n
    s_col = a_w_ref[...] / jnp.sqrt(var_n + EPS)
    t_col = a_b_ref[...] - mean_n * s_col

    w2a = w2a_ref[...]
    s_row = _dot(s_col, eye, ((0,), (0,)))
    mean_row = _dot(mean_n, eye, ((0,), (0,)))
    w2a_s = w2a * s_row
    mean_y = _dot(w2a, s_col * mean_n + t_col, ((1,), (0,)))
    cov = m_sum / CNT - _dot(mean_n, mean_row, ((1,), (0,)))
    v_half = _dot(w2a_s, cov, ((1,), (0,)))
    var_y = jnp.sum(v_half * w2a_s, axis=1, keepdims=True)
    inv_y = 1.0 / jnp.sqrt(var_y + EPS)
    b_w = b_w_ref[...]
    b_b = b_b_ref[...]

    nb_all = jnp.concatenate(
        [nb_ref[h, :, kk, :] for kk in range(KN) for h in range(TN // TNW)],
        axis=1)
    xn = nb_all * s_col + t_col
    y = _dotd(w2a, xn, ((1,), (0,)))
    z = (y - mean_y) * inv_y * b_w + b_b
    g = 0.5 * z * (1.0 + lax.erf(z * (2.0 ** -0.5)))
    h2 = _dotd(w2b_ref[...], g, ((1,), (0,)))
    acc = h2[:, :TN]
    for kk in range(1, KN):
        acc = jnp.maximum(acc, h2[:, kk * TN:(kk + 1) * TN])

    out = (_dotd(fp_ref[...], pe_ref[0], ((1,), (0,))) +
           _dotd(fn_ref[...], acc, ((1,), (0,))) + fb_ref[...])
    out_ref[0] = out


@functools.partial(jax.jit, static_argnames=())
def kernel(x, bn0_w, bn0_b, conv1_w, conv1_b, bn2a_w, bn2a_b, conv2a_w,
           bn2b_w, bn2b_b, conv2b_w, final_w, final_b):
    f32 = jnp.float32
    w0 = bn0_w.reshape(C, 1).astype(f32)
    b0 = bn0_b.reshape(C, 1).astype(f32)
    b1 = conv1_b.reshape(D, 1).astype(f32)
    a_w = bn2a_w.reshape(C, 1).astype(f32)
    a_b = bn2a_b.reshape(C, 1).astype(f32)
    b_w = bn2b_w.reshape(D, 1).astype(f32)
    b_b = bn2b_b.reshape(D, 1).astype(f32)
    fb = final_b.reshape(D, 1).astype(f32)
    fp = final_w[:, :D]
    fn = final_w[:, D:]

    n_tq = N // TQ
    pe, xh, nm = pl.pallas_call(
        _ka_body,
        grid=(B, n_tq),
        in_specs=[
            pl.BlockSpec((B, C, N), lambda b, t: (0, 0, 0)),
            pl.BlockSpec((C, 1), lambda b, t: (0, 0)),
            pl.BlockSpec((C, 1), lambda b, t: (0, 0)),
            pl.BlockSpec((D, C), lambda b, t: (0, 0)),
            pl.BlockSpec((D, 1), lambda b, t: (0, 0)),
        ],
        out_specs=[
            pl.BlockSpec((1, D, TQ), lambda b, t: (b, 0, t)),
            pl.BlockSpec((1, C, TQ), lambda b, t: (b, 0, t)),
            pl.BlockSpec((1, TQ, N), lambda b, t: (b, t, 0)),
        ],
        out_shape=[
            jax.ShapeDtypeStruct((B, D, N), f32),
            jax.ShapeDtypeStruct((B, C, N), f32),
            jax.ShapeDtypeStruct((B, N, N), f32),
        ],
    )(x, w0, b0, conv1_w, b1)

    mesh = plsc.VectorSubcoreMesh(core_axis_name="c", subcore_axis_name="s")
    nb, mom = pl.kernel(
        _sc_body,
        out_type=[
            jax.ShapeDtypeStruct((NW, C, KN, TNW), f32),
            jax.ShapeDtypeStruct((NW, C, 16), f32),
        ],
        mesh=mesh,
        compiler_params=pltpu.CompilerParams(needs_layout_passes=False,
                                             use_tc_tiling_on_sc=False),
        scratch_types=[
            pltpu.VMEM((B * C * N,), f32),
            pltpu.VMEM((QL * N,), f32),
            pltpu.VMEM((NBLK, 16), f32),
            pltpu.VMEM((K, 16), jnp.int32),
            pltpu.VMEM((C, KN, TNW), f32),
            pltpu.VMEM((C, 16), f32),
        ],
    )(nm.reshape(B * N * N), xh.reshape(B * C * N))

    n_tn = N // TN
    out = pl.pallas_call(
        _kb_body,
        grid=(B, n_tn),
        in_specs=[
            pl.BlockSpec((TN // TNW, C, KN, TNW),
                         lambda b, t: (b * (N // TN) + t, 0, 0, 0)),
            pl.BlockSpec((1, D, TN), lambda b, t: (b, 0, t)),
            pl.BlockSpec((NW, C, 16), lambda b, t: (0, 0, 0)),
            pl.BlockSpec((D, C), lambda b, t: (0, 0)),
            pl.BlockSpec((C, 1), lambda b, t: (0, 0)),
            pl.BlockSpec((C, 1), lambda b, t: (0, 0)),
            pl.BlockSpec((D, 1), lambda b, t: (0, 0)),
            pl.BlockSpec((D, 1), lambda b, t: (0, 0)),
            pl.BlockSpec((D, D), lambda b, t: (0, 0)),
            pl.BlockSpec((D, D), lambda b, t: (0, 0)),
            pl.BlockSpec((D, D), lambda b, t: (0, 0)),
            pl.BlockSpec((D, 1), lambda b, t: (0, 0)),
        ],
        out_specs=pl.BlockSpec((1, D, TN), lambda b, t: (b, 0, t)),
        out_shape=jax.ShapeDtypeStruct((B, D, N), f32),
    )(nb, pe, mom, conv2a_w, a_w, a_b, b_w, b_b, conv2b_w, fp, fn, fb)
    return out
